# baseline (speedup 1.0000x reference)
import functools

import jax
import jax.numpy as jnp
from jax import lax
from jax.experimental import pallas as pl
from jax.experimental.pallas import tpu as pltpu
from jax.experimental.pallas import tpu_sc as plsc

NUM_ROWS = 1000000
DIM = 32

NC = 2
NS = 16
NW = NC * NS

BATCH = 16384
SEQ = 50
BW = BATCH // NW
JV = BW // 16
CT = BW // 128


def _body(idx_hbm, table_hbm, out_hbm, idx_v, idxs_v, rows_v, trans_v,
          gsem, psem):
    wid = lax.axis_index("s") * NC + lax.axis_index("c")
    b0 = wid * BW

    pltpu.sync_copy(idx_hbm.at[pl.ds(b0, BW), :], idx_v)

    iota = lax.iota(jnp.int32, 16)

    def repack(s, ib):
        col = jnp.full((16,), 0, jnp.int32) + s

        @plsc.parallel_loop(0, JV, unroll=4)
        def _jv(jv):
            row = iota + (jv * 16)
            vec = plsc.load_gather(idx_v, [row, col])
            idxs_v[ib, pl.ds(jv * 16, 16)] = vec

    def gather(ib, rb):
        src = table_hbm.at[idxs_v.at[ib]]
        return pltpu.make_async_copy(src, rows_v.at[rb], gsem.at[rb])

    def writeback(s, tb):
        dst = out_hbm.at[s, :, pl.ds(CT * wid, CT), :, :]
        return pltpu.make_async_copy(trans_v.at[tb], dst, psem.at[tb])

    def transpose(rb, tb):
        rows = rows_v.at[rb]
        trans = trans_v.at[tb]

        @plsc.parallel_loop(0, DIM, unroll=4)
        def _d(d):
            dt = d // 8
            dp = d - dt * 8
            col = jnp.full((16,), 0, jnp.int32) + d
            for jv in range(JV):
                row = iota + (jv * 16)
                vec = plsc.load_gather(rows, [row, col])
                trans[dt, jv // 8, dp, pl.ds((jv % 8) * 16, 16)] = vec

    repack(0, 0)
    gather(0, 0).start()

    @pl.loop(0, SEQ, step=2)
    def _s2(s0):
        for h in range(2):
            s = s0 + h
            rb = h
            tb = h
            ib = h
            gather(ib, rb).wait()

            @pl.when(s + 1 < SEQ)
            def _():
                repack(s + 1, 1 - ib)
                gather(1 - ib, 1 - rb).start()

            @pl.when(s >= 2)
            def _():
                writeback(s - 2, tb).wait()

            transpose(rb, tb)
            writeback(s, tb).start()

    for tb in range(2):
        writeback(0, tb).wait()


CW = 512
NCHW = NUM_ROWS // CW
CWT = NUM_ROWS - NCHW * CW
TAIL0 = NCHW * CW


def _repack_body(wt_hbm, out_hbm, buf_v, tb_v, bt_v, tt_v):
    wid = lax.axis_index("s") * NC + lax.axis_index("c")
    iota32 = lax.iota(jnp.int32, 16) * DIM

    @pl.loop(0, (NCHW + NW - 1) // NW)
    def _j(i):
        j = i * NW + wid

        @pl.when(j < NCHW)
        def _():
            c0 = pl.multiple_of(j * CW, CW)
            pltpu.sync_copy(wt_hbm.at[:, pl.ds(c0, CW)], buf_v)

            @plsc.parallel_loop(0, DIM, unroll=4)
            def _d(d):
                for jv in range(CW // 16):
                    vec = buf_v[d, pl.ds(jv * 16, 16)]
                    idx = iota32 + (jv * 16 * DIM + d)
                    plsc.store_scatter(tb_v, [idx], vec)

            pltpu.sync_copy(tb_v, out_hbm.at[pl.ds(c0 * DIM, CW * DIM)])

    @pl.when(wid == 0)
    def _tail():
        pltpu.sync_copy(wt_hbm.at[:, pl.ds(TAIL0, CWT)], bt_v)

        @plsc.parallel_loop(0, DIM, unroll=4)
        def _d(d):
            for jv in range(CWT // 16):
                vec = bt_v[d, pl.ds(jv * 16, 16)]
                idx = iota32 + (jv * 16 * DIM + d)
                plsc.store_scatter(tt_v, [idx], vec)

        pltpu.sync_copy(tt_v, out_hbm.at[pl.ds(TAIL0 * DIM, CWT * DIM)])


@jax.jit
def _repack(weight_t):
    mesh = plsc.VectorSubcoreMesh(core_axis_name="c", subcore_axis_name="s")
    f = functools.partial(
        pl.kernel,
        out_type=jax.ShapeDtypeStruct((NUM_ROWS * DIM,), jnp.float32),
        mesh=mesh,
        scratch_types=[
            pltpu.VMEM((DIM, CW), jnp.float32),
            pltpu.VMEM((CW * DIM,), jnp.float32),
            pltpu.VMEM((DIM, CWT), jnp.float32),
            pltpu.VMEM((CWT * DIM,), jnp.float32),
        ],
        compiler_params=pltpu.CompilerParams(
            use_tc_tiling_on_sc=True, needs_layout_passes=False
        ),
    )(_repack_body)
    return f(weight_t)


@jax.jit
def _lookup(token_ids, weight):
    mesh = plsc.VectorSubcoreMesh(core_axis_name="c", subcore_axis_name="s")
    f = functools.partial(
        pl.kernel,
        out_type=jax.ShapeDtypeStruct(
            (SEQ, DIM // 8, BATCH // 128, 8, 128), jnp.float32
        ),
        mesh=mesh,
        scratch_types=[
            pltpu.VMEM((BW, SEQ), jnp.int32),
            pltpu.VMEM((2, BW), jnp.int32),
            pltpu.VMEM((2, BW, DIM), jnp.float32),
            pltpu.VMEM((2, DIM // 8, CT, 8, 128), jnp.float32),
            pltpu.SemaphoreType.DMA((2,)),
            pltpu.SemaphoreType.DMA((2,)),
        ],
        compiler_params=pltpu.CompilerParams(
            use_tc_tiling_on_sc=False, needs_layout_passes=False
        ),
    )(_body)
    return f(token_ids, weight)


def kernel(token_ids, weight):
    wlin = _repack(weight.T)
    out5 = _lookup(token_ids.astype(jnp.int32), wlin.reshape(NUM_ROWS, DIM))
    return out5.transpose(2, 4, 0, 1, 3).reshape(BATCH, SEQ, DIM)


# speedup vs baseline: 1.1593x; 1.1593x over previous
import functools

import jax
import jax.numpy as jnp
from jax import lax
from jax.experimental import pallas as pl
from jax.experimental.pallas import tpu as pltpu
from jax.experimental.pallas import tpu_sc as plsc

NUM_ROWS = 1000000
DIM = 32

NC = 2
NS = 16
NW = NC * NS

BATCH = 16384
SEQ = 50
BW = BATCH // NW
JV = BW // 16
CT = BW // 128


def _body(idx_hbm, table_hbm, out_hbm, idx_v, idxs_v, rows_v, trans_v,
          gsem, psem):
    wid = lax.axis_index("s") * NC + lax.axis_index("c")
    b0 = wid * BW

    pltpu.sync_copy(idx_hbm.at[pl.ds(b0, BW), :], idx_v)

    iota = lax.iota(jnp.int32, 16)

    def repack(s, ib):
        col = jnp.full((16,), 0, jnp.int32) + s

        @plsc.parallel_loop(0, JV, unroll=4)
        def _jv(jv):
            row = iota + (jv * 16)
            vec = plsc.load_gather(idx_v, [row, col])
            idxs_v[ib, pl.ds(jv * 16, 16)] = vec

    def gather(ib, rb):
        src = table_hbm.at[idxs_v.at[ib]]
        return pltpu.make_async_copy(src, rows_v.at[rb], gsem.at[rb])

    def writeback(s, tb):
        dst = out_hbm.at[s, :, pl.ds(CT * wid, CT), :, :]
        return pltpu.make_async_copy(trans_v.at[tb], dst, psem.at[tb])

    def transpose(rb, tb):
        rows = rows_v.at[rb]
        trans = trans_v.at[tb]

        @plsc.parallel_loop(0, DIM, unroll=4)
        def _d(d):
            dt = d // 8
            dp = d - dt * 8
            col = jnp.full((16,), 0, jnp.int32) + d
            for jv in range(JV):
                row = iota + (jv * 16)
                vec = plsc.load_gather(rows, [row, col])
                trans[dt, jv // 8, dp, pl.ds((jv % 8) * 16, 16)] = vec

    repack(0, 0)
    gather(0, 0).start()

    @pl.loop(0, SEQ, step=2)
    def _s2(s0):
        for h in range(2):
            s = s0 + h
            rb = h
            tb = h
            ib = h
            gather(ib, rb).wait()

            @pl.when(s + 1 < SEQ)
            def _():
                repack(s + 1, 1 - ib)
                gather(1 - ib, 1 - rb).start()

            @pl.when(s >= 2)
            def _():
                writeback(s - 2, tb).wait()

            transpose(rb, tb)
            writeback(s, tb).start()

    for tb in range(2):
        writeback(0, tb).wait()


CW = 512
NCHW = NUM_ROWS // CW
CWT = NUM_ROWS - NCHW * CW
TAIL0 = NCHW * CW


PC = (NCHW + NW - 1) // NW


def _repack_body(wt_hbm, out_hbm, buf_a, buf_b, tb_a, tb_b, bt_v, tt_v,
                 ism, osm):
    bufs = (buf_a, buf_b)
    tbs = (tb_a, tb_b)
    wid = lax.axis_index("s") * NC + lax.axis_index("c")
    iota32 = lax.iota(jnp.int32, 16) * DIM

    def cstart(i):
        return pl.multiple_of((i * NW + wid) * CW, CW)

    def copy_in(i, b):
        src = wt_hbm.at[:, pl.ds(cstart(i), CW)]
        return pltpu.make_async_copy(src, bufs[b], ism.at[b])

    def copy_out(i, b):
        dst = out_hbm.at[pl.ds(cstart(i) * DIM, CW * DIM)]
        return pltpu.make_async_copy(tbs[b], dst, osm.at[b])

    def valid(i):
        return (i * NW + wid) < NCHW

    def transpose(b):
        @plsc.parallel_loop(0, DIM, unroll=4)
        def _d(d):
            for jv in range(CW // 16):
                vec = bufs[b][d, pl.ds(jv * 16, 16)]
                idx = iota32 + (jv * 16 * DIM + d)
                plsc.store_scatter(tbs[b], [idx], vec)

    @pl.when(valid(0))
    def _p():
        copy_in(0, 0).start()

    @pl.loop(0, PC, step=2)
    def _i2(i0):
        for h in range(2):
            i = i0 + h
            b = h

            @pl.when(valid(i))
            def _():
                copy_in(i, b).wait()

                @pl.when(valid(i + 1))
                def _():
                    copy_in(i + 1, 1 - b).start()

                @pl.when(i >= 2)
                def _():
                    copy_out(0, b).wait()

                transpose(b)
                copy_out(i, b).start()

    for b in range(2):
        copy_out(0, b).wait()

    @pl.when(wid == 0)
    def _tail():
        pltpu.sync_copy(wt_hbm.at[:, pl.ds(TAIL0, CWT)], bt_v)

        @plsc.parallel_loop(0, DIM, unroll=4)
        def _d(d):
            for jv in range(CWT // 16):
                vec = bt_v[d, pl.ds(jv * 16, 16)]
                idx = iota32 + (jv * 16 * DIM + d)
                plsc.store_scatter(tt_v, [idx], vec)

        pltpu.sync_copy(tt_v, out_hbm.at[pl.ds(TAIL0 * DIM, CWT * DIM)])


@jax.jit
def _repack(weight_t):
    mesh = plsc.VectorSubcoreMesh(core_axis_name="c", subcore_axis_name="s")
    f = functools.partial(
        pl.kernel,
        out_type=jax.ShapeDtypeStruct((NUM_ROWS * DIM,), jnp.float32),
        mesh=mesh,
        scratch_types=[
            pltpu.VMEM((DIM, CW), jnp.float32),
            pltpu.VMEM((DIM, CW), jnp.float32),
            pltpu.VMEM((CW * DIM,), jnp.float32),
            pltpu.VMEM((CW * DIM,), jnp.float32),
            pltpu.VMEM((DIM, CWT), jnp.float32),
            pltpu.VMEM((CWT * DIM,), jnp.float32),
            pltpu.SemaphoreType.DMA((2,)),
            pltpu.SemaphoreType.DMA((2,)),
        ],
        compiler_params=pltpu.CompilerParams(
            use_tc_tiling_on_sc=True, needs_layout_passes=False
        ),
    )(_repack_body)
    return f(weight_t)


@jax.jit
def _lookup(token_ids, weight):
    mesh = plsc.VectorSubcoreMesh(core_axis_name="c", subcore_axis_name="s")
    f = functools.partial(
        pl.kernel,
        out_type=jax.ShapeDtypeStruct(
            (SEQ, DIM // 8, BATCH // 128, 8, 128), jnp.float32
        ),
        mesh=mesh,
        scratch_types=[
            pltpu.VMEM((BW, SEQ), jnp.int32),
            pltpu.VMEM((2, BW), jnp.int32),
            pltpu.VMEM((2, BW, DIM), jnp.float32),
            pltpu.VMEM((2, DIM // 8, CT, 8, 128), jnp.float32),
            pltpu.SemaphoreType.DMA((2,)),
            pltpu.SemaphoreType.DMA((2,)),
        ],
        compiler_params=pltpu.CompilerParams(
            use_tc_tiling_on_sc=False, needs_layout_passes=False
        ),
    )(_body)
    return f(token_ids, weight)


def kernel(token_ids, weight):
    wlin = _repack(weight.T)
    out5 = _lookup(token_ids.astype(jnp.int32), wlin.reshape(NUM_ROWS, DIM))
    return out5.transpose(2, 4, 0, 1, 3).reshape(BATCH, SEQ, DIM)
